# baseline (speedup 1.0000x reference)
import functools

import jax
import jax.numpy as jnp
from jax import lax
from jax.experimental import pallas as pl
from jax.experimental.pallas import tpu as pltpu
from jax.experimental.pallas import tpu_sc as plsc

N = 10000
E = 320000
F = 128
H = 32
EMB = 16

NC = 2
NS = 16
NW = NC * NS
B = 128
G = 80
E_PAD = NW * G * B
NBUF = 4

N_ACC = 10112
ZCHUNK = N_ACC // NS

_mesh = plsc.VectorSubcoreMesh(core_axis_name="c", subcore_axis_name="s")
_sc_params = pltpu.CompilerParams(use_tc_tiling_on_sc=False)


def _deg_body(col_hbm, ones_hbm, zeros_hbm, out_hbm, colv, onesv, shared,
              ssems):
    c = lax.axis_index("c")
    s = lax.axis_index("s")
    wid = c * NS + s
    pltpu.sync_copy(zeros_hbm.at[pl.ds(s * ZCHUNK, ZCHUNK)],
                    shared.at[pl.ds(s * ZCHUNK, ZCHUNK)])
    pltpu.sync_copy(ones_hbm, onesv)
    pltpu.sync_copy(col_hbm.at[pl.ds(wid * G, G)], colv)
    plsc.subcore_barrier()

    @pl.loop(0, G, step=NBUF)
    def _(g0):
        for b in range(NBUF):
            g = g0 + b

            @pl.when(g >= NBUF)
            def _drain():
                pltpu.make_async_copy(onesv, shared.at[colv.at[g - NBUF]],
                                      ssems.at[b]).wait()

            pltpu.async_copy(onesv, shared.at[colv.at[g]], ssems.at[b],
                             add=True)

    for b in range(NBUF):
        pltpu.make_async_copy(onesv, shared.at[colv.at[G - NBUF + b]],
                              ssems.at[b]).wait()

    plsc.subcore_barrier()
    pltpu.sync_copy(shared.at[pl.ds(s * ZCHUNK, ZCHUNK)],
                    out_hbm.at[c, pl.ds(s * ZCHUNK, ZCHUNK)])


_deg_kernel = functools.partial(
    pl.kernel,
    out_type=jax.ShapeDtypeStruct((NC, N_ACC, 16), jnp.float32),
    mesh=_mesh,
    compiler_params=_sc_params,
    scratch_types=[
        pltpu.VMEM((G, B), jnp.int32),
        pltpu.VMEM((B, 16), jnp.float32),
        pltpu.VMEM_SHARED((N_ACC, 16), jnp.float32),
        pltpu.SemaphoreType.DMA((NBUF,)),
    ],
)(_deg_body)


def _agg_body(row_hbm, col_hbm, hs_hbm, zeros_hbm, out_hbm,
              rowv, colv, bufs, shared, gsems, ssems):
    c = lax.axis_index("c")
    s = lax.axis_index("s")
    wid = c * NS + s
    pltpu.sync_copy(zeros_hbm.at[pl.ds(s * ZCHUNK, ZCHUNK)],
                    shared.at[pl.ds(s * ZCHUNK, ZCHUNK)])
    pltpu.sync_copy(row_hbm.at[pl.ds(wid * G, G)], rowv)
    pltpu.sync_copy(col_hbm.at[pl.ds(wid * G, G)], colv)
    plsc.subcore_barrier()

    NB2 = 2 * NBUF
    for b in range(NBUF):
        pltpu.async_copy(hs_hbm.at[rowv.at[b]], bufs.at[b], gsems.at[b])

    @pl.loop(0, G, step=NB2)
    def _(g0):
        for db in range(NB2):
            g = g0 + db
            b = db
            pltpu.make_async_copy(hs_hbm.at[rowv.at[g]], bufs.at[b],
                                  gsems.at[b]).wait()
            pltpu.async_copy(bufs.at[b], shared.at[colv.at[g]], ssems.at[b],
                             add=True)
            gn = g + NBUF
            bn = (db + NBUF) % NB2

            @pl.when(gn < G)
            def _issue():
                @pl.when(gn >= NB2)
                def _free():
                    pltpu.make_async_copy(bufs.at[bn],
                                          shared.at[colv.at[gn - NB2]],
                                          ssems.at[bn]).wait()

                pltpu.async_copy(hs_hbm.at[rowv.at[gn]], bufs.at[bn],
                                 gsems.at[bn])

    for b in range(NB2):
        g_last = G - NB2 + b
        pltpu.make_async_copy(bufs.at[b], shared.at[colv.at[g_last]],
                              ssems.at[b]).wait()

    plsc.subcore_barrier()
    pltpu.sync_copy(shared.at[pl.ds(s * ZCHUNK, ZCHUNK)],
                    out_hbm.at[c, pl.ds(s * ZCHUNK, ZCHUNK)])


def _make_agg(D):
    return functools.partial(
        pl.kernel,
        out_type=jax.ShapeDtypeStruct((NC, N_ACC, D), jnp.float32),
        mesh=_mesh,
        compiler_params=_sc_params,
        scratch_types=[
            pltpu.VMEM((G, B), jnp.int32),
            pltpu.VMEM((G, B), jnp.int32),
            pltpu.VMEM((2 * NBUF, B, D), jnp.float32),
            pltpu.VMEM_SHARED((N_ACC, D), jnp.float32),
            pltpu.SemaphoreType.DMA((2 * NBUF,)),
            pltpu.SemaphoreType.DMA((2 * NBUF,)),
        ],
    )(_agg_body)


_agg32 = _make_agg(H)
_agg16 = _make_agg(EMB)

_RB = 1000


def _tc_a_body(x_ref, w_ref, da_ref, db_ref, hs_ref, dinv_ref):
    deg = da_ref[...] + db_ref[...] + 1.0
    dinv = lax.rsqrt(deg)
    h = jnp.dot(x_ref[...], w_ref[...], preferred_element_type=jnp.float32)
    hs_ref[...] = h * dinv
    dinv_ref[...] = dinv


def _tc_a(x, w1, dega, degb):
    return pl.pallas_call(
        _tc_a_body,
        grid=(N // _RB,),
        in_specs=[
            pl.BlockSpec((_RB, F), lambda i: (i, 0)),
            pl.BlockSpec((F, H), lambda i: (0, 0)),
            pl.BlockSpec((_RB, 1), lambda i: (i, 0)),
            pl.BlockSpec((_RB, 1), lambda i: (i, 0)),
        ],
        out_specs=[
            pl.BlockSpec((_RB, H), lambda i: (i, 0)),
            pl.BlockSpec((_RB, 1), lambda i: (i, 0)),
        ],
        out_shape=[
            jax.ShapeDtypeStruct((N, H), jnp.float32),
            jax.ShapeDtypeStruct((N, 1), jnp.float32),
        ],
    )(x, w1, dega, degb)


def _tc_b_body(aa_ref, ab_ref, hs_ref, dinv_ref, b1_ref, w2_ref, out_ref):
    dinv = dinv_ref[...]
    pre = dinv * (aa_ref[...] + ab_ref[...] + hs_ref[...]) + b1_ref[...]
    r = jnp.maximum(pre, 0.0)
    h2 = jnp.dot(r, w2_ref[...], preferred_element_type=jnp.float32)
    out_ref[...] = h2 * dinv


def _tc_b(agg_a, agg_b, hs1, dinv, b1, w2):
    return pl.pallas_call(
        _tc_b_body,
        grid=(N // _RB,),
        in_specs=[
            pl.BlockSpec((_RB, H), lambda i: (i, 0)),
            pl.BlockSpec((_RB, H), lambda i: (i, 0)),
            pl.BlockSpec((_RB, H), lambda i: (i, 0)),
            pl.BlockSpec((_RB, 1), lambda i: (i, 0)),
            pl.BlockSpec((1, H), lambda i: (0, 0)),
            pl.BlockSpec((H, EMB), lambda i: (0, 0)),
        ],
        out_specs=pl.BlockSpec((_RB, EMB), lambda i: (i, 0)),
        out_shape=jax.ShapeDtypeStruct((N, EMB), jnp.float32),
    )(agg_a, agg_b, hs1, dinv, b1, w2)


def _tc_c_body(aa_ref, ab_ref, hs_ref, dinv_ref, b2_ref, out_ref):
    out_ref[...] = (dinv_ref[...] *
                    (aa_ref[...] + ab_ref[...] + hs_ref[...]) + b2_ref[...])


def _tc_c(agg_a, agg_b, hs2, dinv, b2):
    return pl.pallas_call(
        _tc_c_body,
        grid=(N // _RB,),
        in_specs=[
            pl.BlockSpec((_RB, EMB), lambda i: (i, 0)),
            pl.BlockSpec((_RB, EMB), lambda i: (i, 0)),
            pl.BlockSpec((_RB, EMB), lambda i: (i, 0)),
            pl.BlockSpec((_RB, 1), lambda i: (i, 0)),
            pl.BlockSpec((1, EMB), lambda i: (0, 0)),
        ],
        out_specs=pl.BlockSpec((_RB, EMB), lambda i: (i, 0)),
        out_shape=jax.ShapeDtypeStruct((N, EMB), jnp.float32),
    )(agg_a, agg_b, hs2, dinv, b2)


def kernel(x, ei, W1, b1, W2, b2):
    ei = ei.astype(jnp.int32)
    pad = E_PAD - E
    row_r = jnp.concatenate(
        [ei[0], jnp.zeros((pad,), jnp.int32)]).reshape(NW * G, B)
    col_r = jnp.concatenate(
        [ei[1], jnp.full((pad,), N, jnp.int32)]).reshape(NW * G, B)

    ones16 = jnp.ones((B, 16), jnp.float32)
    zeros16 = jnp.zeros((N_ACC, 16), jnp.float32)
    zeros32 = jnp.zeros((N_ACC, H), jnp.float32)

    deg2d = _deg_kernel(col_r, ones16, zeros16)
    dega = deg2d[0, :N, 0:1]
    degb = deg2d[1, :N, 0:1]

    hs1, dinv = _tc_a(x, W1, dega, degb)

    agg1 = _agg32(row_r, col_r, hs1, zeros32)
    hs2 = _tc_b(agg1[0, :N], agg1[1, :N], hs1, dinv, b1.reshape(1, H), W2)

    agg2 = _agg16(row_r, col_r, hs2, zeros16)
    out = _tc_c(agg2[0, :N], agg2[1, :N], hs2, dinv, b2.reshape(1, EMB))
    return out


# speedup vs baseline: 1.3404x; 1.3404x over previous
import functools

import jax
import jax.numpy as jnp
from jax import lax
from jax.experimental import pallas as pl
from jax.experimental.pallas import tpu as pltpu
from jax.experimental.pallas import tpu_sc as plsc

N = 10000
E = 320000
F = 128
H = 32
EMB = 16

NC = 2
NS = 16
NW = NC * NS
B = 128
G = 80
E_PAD = NW * G * B
NBUF = 4

N_ACC = 10112
ZCHUNK = N_ACC // NS

_mesh = plsc.VectorSubcoreMesh(core_axis_name="c", subcore_axis_name="s")
_sc_params = pltpu.CompilerParams(use_tc_tiling_on_sc=False)


def _deg_body(col_hbm, ones_hbm, zeros_hbm, out_hbm, colv, onesv, shared,
              ssems):
    c = lax.axis_index("c")
    s = lax.axis_index("s")
    wid = c * NS + s
    pltpu.sync_copy(zeros_hbm.at[pl.ds(s * ZCHUNK, ZCHUNK)],
                    shared.at[pl.ds(s * ZCHUNK, ZCHUNK)])
    pltpu.sync_copy(ones_hbm, onesv)
    pltpu.sync_copy(col_hbm.at[pl.ds(wid * G, G)], colv)
    plsc.subcore_barrier()

    @pl.loop(0, G, step=NBUF)
    def _(g0):
        for b in range(NBUF):
            g = g0 + b

            @pl.when(g >= NBUF)
            def _drain():
                pltpu.make_async_copy(onesv, shared.at[colv.at[g - NBUF]],
                                      ssems.at[b]).wait()

            pltpu.async_copy(onesv, shared.at[colv.at[g]], ssems.at[b],
                             add=True)

    for b in range(NBUF):
        pltpu.make_async_copy(onesv, shared.at[colv.at[G - NBUF + b]],
                              ssems.at[b]).wait()

    plsc.subcore_barrier()
    pltpu.sync_copy(shared.at[pl.ds(s * ZCHUNK, ZCHUNK)],
                    out_hbm.at[c, pl.ds(s * ZCHUNK, ZCHUNK)])


_deg_kernel = functools.partial(
    pl.kernel,
    out_type=jax.ShapeDtypeStruct((NC, N_ACC, 16), jnp.float32),
    mesh=_mesh,
    compiler_params=_sc_params,
    scratch_types=[
        pltpu.VMEM((G, B), jnp.int32),
        pltpu.VMEM((B, 16), jnp.float32),
        pltpu.VMEM_SHARED((N_ACC, 16), jnp.float32),
        pltpu.SemaphoreType.DMA((NBUF,)),
    ],
)(_deg_body)


def _agg_body(row_hbm, col_hbm, hs_hbm, zeros_hbm, out_hbm,
              rowv, colv, bufs, shared, gsems, ssems, table=None):
    c = lax.axis_index("c")
    s = lax.axis_index("s")
    wid = c * NS + s
    pltpu.sync_copy(zeros_hbm.at[pl.ds(s * ZCHUNK, ZCHUNK)],
                    shared.at[pl.ds(s * ZCHUNK, ZCHUNK)])
    if table is not None:
        pltpu.sync_copy(hs_hbm.at[pl.ds(s * (N // NS), N // NS)],
                        table.at[pl.ds(s * (N // NS), N // NS)])
        src = table
    else:
        src = hs_hbm
    pltpu.sync_copy(row_hbm.at[pl.ds(wid * G, G)], rowv)
    pltpu.sync_copy(col_hbm.at[pl.ds(wid * G, G)], colv)
    plsc.subcore_barrier()

    NB2 = 2 * NBUF
    for b in range(NBUF):
        pltpu.async_copy(src.at[rowv.at[b]], bufs.at[b], gsems.at[b])

    @pl.loop(0, G, step=NB2)
    def _(g0):
        for db in range(NB2):
            g = g0 + db
            b = db
            pltpu.make_async_copy(src.at[rowv.at[g]], bufs.at[b],
                                  gsems.at[b]).wait()
            pltpu.async_copy(bufs.at[b], shared.at[colv.at[g]], ssems.at[b],
                             add=True)
            gn = g + NBUF
            bn = (db + NBUF) % NB2

            @pl.when(gn < G)
            def _issue():
                @pl.when(gn >= NB2)
                def _free():
                    pltpu.make_async_copy(bufs.at[bn],
                                          shared.at[colv.at[gn - NB2]],
                                          ssems.at[bn]).wait()

                pltpu.async_copy(src.at[rowv.at[gn]], bufs.at[bn],
                                 gsems.at[bn])

    for b in range(NB2):
        g_last = G - NB2 + b
        pltpu.make_async_copy(bufs.at[b], shared.at[colv.at[g_last]],
                              ssems.at[b]).wait()

    plsc.subcore_barrier()
    pltpu.sync_copy(shared.at[pl.ds(s * ZCHUNK, ZCHUNK)],
                    out_hbm.at[c, pl.ds(s * ZCHUNK, ZCHUNK)])


def _make_agg(D, spmem_table):
    scratch = [
        pltpu.VMEM((G, B), jnp.int32),
        pltpu.VMEM((G, B), jnp.int32),
        pltpu.VMEM((2 * NBUF, B, D), jnp.float32),
        pltpu.VMEM_SHARED((N_ACC, D), jnp.float32),
        pltpu.SemaphoreType.DMA((2 * NBUF,)),
        pltpu.SemaphoreType.DMA((2 * NBUF,)),
    ]
    if spmem_table:
        scratch.append(pltpu.VMEM_SHARED((N, D), jnp.float32))
    return functools.partial(
        pl.kernel,
        out_type=jax.ShapeDtypeStruct((NC, N_ACC, D), jnp.float32),
        mesh=_mesh,
        compiler_params=_sc_params,
        scratch_types=scratch,
    )(_agg_body)


_agg32 = _make_agg(H, True)
_agg16 = _make_agg(EMB, False)

_RB = 1000


def _tc_a_body(x_ref, w_ref, da_ref, db_ref, hs_ref, dinv_ref):
    deg = da_ref[...] + db_ref[...] + 1.0
    dinv = lax.rsqrt(deg)
    h = jnp.dot(x_ref[...], w_ref[...], preferred_element_type=jnp.float32)
    hs_ref[...] = h * dinv
    dinv_ref[...] = dinv


def _tc_a(x, w1, dega, degb):
    return pl.pallas_call(
        _tc_a_body,
        grid=(N // _RB,),
        in_specs=[
            pl.BlockSpec((_RB, F), lambda i: (i, 0)),
            pl.BlockSpec((F, H), lambda i: (0, 0)),
            pl.BlockSpec((_RB, 1), lambda i: (i, 0)),
            pl.BlockSpec((_RB, 1), lambda i: (i, 0)),
        ],
        out_specs=[
            pl.BlockSpec((_RB, H), lambda i: (i, 0)),
            pl.BlockSpec((_RB, 1), lambda i: (i, 0)),
        ],
        out_shape=[
            jax.ShapeDtypeStruct((N, H), jnp.float32),
            jax.ShapeDtypeStruct((N, 1), jnp.float32),
        ],
    )(x, w1, dega, degb)


def _tc_b_body(aa_ref, ab_ref, hs_ref, dinv_ref, b1_ref, w2_ref, out_ref):
    dinv = dinv_ref[...]
    pre = dinv * (aa_ref[...] + ab_ref[...] + hs_ref[...]) + b1_ref[...]
    r = jnp.maximum(pre, 0.0)
    h2 = jnp.dot(r, w2_ref[...], preferred_element_type=jnp.float32)
    out_ref[...] = h2 * dinv


def _tc_b(agg_a, agg_b, hs1, dinv, b1, w2):
    return pl.pallas_call(
        _tc_b_body,
        grid=(N // _RB,),
        in_specs=[
            pl.BlockSpec((_RB, H), lambda i: (i, 0)),
            pl.BlockSpec((_RB, H), lambda i: (i, 0)),
            pl.BlockSpec((_RB, H), lambda i: (i, 0)),
            pl.BlockSpec((_RB, 1), lambda i: (i, 0)),
            pl.BlockSpec((1, H), lambda i: (0, 0)),
            pl.BlockSpec((H, EMB), lambda i: (0, 0)),
        ],
        out_specs=pl.BlockSpec((_RB, EMB), lambda i: (i, 0)),
        out_shape=jax.ShapeDtypeStruct((N, EMB), jnp.float32),
    )(agg_a, agg_b, hs1, dinv, b1, w2)


def _tc_c_body(aa_ref, ab_ref, hs_ref, dinv_ref, b2_ref, out_ref):
    out_ref[...] = (dinv_ref[...] *
                    (aa_ref[...] + ab_ref[...] + hs_ref[...]) + b2_ref[...])


def _tc_c(agg_a, agg_b, hs2, dinv, b2):
    return pl.pallas_call(
        _tc_c_body,
        grid=(N // _RB,),
        in_specs=[
            pl.BlockSpec((_RB, EMB), lambda i: (i, 0)),
            pl.BlockSpec((_RB, EMB), lambda i: (i, 0)),
            pl.BlockSpec((_RB, EMB), lambda i: (i, 0)),
            pl.BlockSpec((_RB, 1), lambda i: (i, 0)),
            pl.BlockSpec((1, EMB), lambda i: (0, 0)),
        ],
        out_specs=pl.BlockSpec((_RB, EMB), lambda i: (i, 0)),
        out_shape=jax.ShapeDtypeStruct((N, EMB), jnp.float32),
    )(agg_a, agg_b, hs2, dinv, b2)


def kernel(x, ei, W1, b1, W2, b2):
    ei = ei.astype(jnp.int32)
    pad = E_PAD - E
    row_r = jnp.concatenate(
        [ei[0], jnp.zeros((pad,), jnp.int32)]).reshape(NW * G, B)
    col_r = jnp.concatenate(
        [ei[1], jnp.full((pad,), N, jnp.int32)]).reshape(NW * G, B)

    ones16 = jnp.ones((B, 16), jnp.float32)
    zeros16 = jnp.zeros((N_ACC, 16), jnp.float32)
    zeros32 = jnp.zeros((N_ACC, H), jnp.float32)

    deg2d = _deg_kernel(col_r, ones16, zeros16)
    dega = deg2d[0, :N, 0:1]
    degb = deg2d[1, :N, 0:1]

    hs1, dinv = _tc_a(x, W1, dega, degb)

    agg1 = _agg32(row_r, col_r, hs1, zeros32)
    hs2 = _tc_b(agg1[0, :N], agg1[1, :N], hs1, dinv, b1.reshape(1, H), W2)

    agg2 = _agg16(row_r, col_r, hs2, zeros16)
    out = _tc_c(agg2[0, :N], agg2[1, :N], hs2, dinv, b2.reshape(1, EMB))
    return out


# speedup vs baseline: 1.6103x; 1.2013x over previous
import functools

import jax
import jax.numpy as jnp
from jax import lax
from jax.experimental import pallas as pl
from jax.experimental.pallas import tpu as pltpu
from jax.experimental.pallas import tpu_sc as plsc

N = 10000
E = 320000
F = 128
H = 32
EMB = 16

NC = 2
NS = 16
NW = NC * NS
B = 128
G = 80
E_PAD = NW * G * B
NBUF = 4

N_ACC = 10112
ZCHUNK = N_ACC // NS

_mesh = plsc.VectorSubcoreMesh(core_axis_name="c", subcore_axis_name="s")
_sc_params = pltpu.CompilerParams(use_tc_tiling_on_sc=False)


def _deg_body(col_hbm, ones_hbm, zeros_hbm, out_hbm, colv, onesv, shared,
              ssems):
    c = lax.axis_index("c")
    s = lax.axis_index("s")
    wid = c * NS + s
    pltpu.sync_copy(zeros_hbm.at[pl.ds(s * ZCHUNK, ZCHUNK)],
                    shared.at[pl.ds(s * ZCHUNK, ZCHUNK)])
    pltpu.sync_copy(ones_hbm, onesv)
    pltpu.sync_copy(col_hbm.at[pl.ds(wid * G, G)], colv)
    plsc.subcore_barrier()

    @pl.loop(0, G, step=NBUF)
    def _(g0):
        for b in range(NBUF):
            g = g0 + b

            @pl.when(g >= NBUF)
            def _drain():
                pltpu.make_async_copy(onesv, shared.at[colv.at[g - NBUF]],
                                      ssems.at[b]).wait()

            pltpu.async_copy(onesv, shared.at[colv.at[g]], ssems.at[b],
                             add=True)

    for b in range(NBUF):
        pltpu.make_async_copy(onesv, shared.at[colv.at[G - NBUF + b]],
                              ssems.at[b]).wait()

    plsc.subcore_barrier()
    pltpu.sync_copy(shared.at[pl.ds(s * ZCHUNK, ZCHUNK)],
                    out_hbm.at[c, pl.ds(s * ZCHUNK, ZCHUNK)])


_deg_kernel = functools.partial(
    pl.kernel,
    out_type=jax.ShapeDtypeStruct((NC, N_ACC, 16), jnp.float32),
    mesh=_mesh,
    compiler_params=_sc_params,
    scratch_types=[
        pltpu.VMEM((G, B), jnp.int32),
        pltpu.VMEM((B, 16), jnp.float32),
        pltpu.VMEM_SHARED((N_ACC, 16), jnp.float32),
        pltpu.SemaphoreType.DMA((NBUF,)),
    ],
)(_deg_body)


def _agg_body(row_hbm, col_hbm, hs_hbm, zeros_hbm, out_hbm,
              rowv, colv, bufs, shared, gsems, ssems, table=None):
    c = lax.axis_index("c")
    s = lax.axis_index("s")
    wid = c * NS + s
    pltpu.sync_copy(zeros_hbm.at[pl.ds(s * ZCHUNK, ZCHUNK)],
                    shared.at[pl.ds(s * ZCHUNK, ZCHUNK)])
    if table is not None:
        pltpu.sync_copy(hs_hbm.at[pl.ds(s * (N // NS), N // NS)],
                        table.at[pl.ds(s * (N // NS), N // NS)])
        src = table
    else:
        src = hs_hbm
    pltpu.sync_copy(row_hbm.at[pl.ds(wid * G, G)], rowv)
    pltpu.sync_copy(col_hbm.at[pl.ds(wid * G, G)], colv)
    plsc.subcore_barrier()

    NB2 = 2 * NBUF
    for b in range(NBUF):
        pltpu.async_copy(src.at[rowv.at[b]], bufs.at[b], gsems.at[b])

    @pl.loop(0, G, step=NB2)
    def _(g0):
        for db in range(NB2):
            g = g0 + db
            b = db
            pltpu.make_async_copy(src.at[rowv.at[g]], bufs.at[b],
                                  gsems.at[b]).wait()
            pltpu.async_copy(bufs.at[b], shared.at[colv.at[g]], ssems.at[b],
                             add=True)
            gn = g + NBUF
            bn = (db + NBUF) % NB2

            @pl.when(gn < G)
            def _issue():
                @pl.when(gn >= NB2)
                def _free():
                    pltpu.make_async_copy(bufs.at[bn],
                                          shared.at[colv.at[gn - NB2]],
                                          ssems.at[bn]).wait()

                pltpu.async_copy(src.at[rowv.at[gn]], bufs.at[bn],
                                 gsems.at[bn])

    for b in range(NB2):
        g_last = G - NB2 + b
        pltpu.make_async_copy(bufs.at[b], shared.at[colv.at[g_last]],
                              ssems.at[b]).wait()

    plsc.subcore_barrier()
    pltpu.sync_copy(shared.at[pl.ds(s * ZCHUNK, ZCHUNK)],
                    out_hbm.at[c, pl.ds(s * ZCHUNK, ZCHUNK)])


def _make_agg(D, spmem_table):
    scratch = [
        pltpu.VMEM((G, B), jnp.int32),
        pltpu.VMEM((G, B), jnp.int32),
        pltpu.VMEM((2 * NBUF, B, D), jnp.float32),
        pltpu.VMEM_SHARED((N_ACC, D), jnp.float32),
        pltpu.SemaphoreType.DMA((2 * NBUF,)),
        pltpu.SemaphoreType.DMA((2 * NBUF,)),
    ]
    if spmem_table:
        scratch.append(pltpu.VMEM_SHARED((N, D), jnp.float32))
    return functools.partial(
        pl.kernel,
        out_type=jax.ShapeDtypeStruct((NC, N_ACC, D), jnp.float32),
        mesh=_mesh,
        compiler_params=_sc_params,
        scratch_types=scratch,
    )(_agg_body)


_agg32 = _make_agg(H, True)
_agg16 = _make_agg(EMB, True)

_RB = 1000


def _tc_a_body(x_ref, w_ref, da_ref, db_ref, hs_ref, dinv_ref):
    deg = da_ref[...] + db_ref[...] + 1.0
    dinv = lax.rsqrt(deg)
    h = jnp.dot(x_ref[...], w_ref[...], preferred_element_type=jnp.float32)
    hs_ref[...] = h * dinv
    dinv_ref[...] = dinv


def _tc_a(x, w1, dega, degb):
    return pl.pallas_call(
        _tc_a_body,
        grid=(N // _RB,),
        in_specs=[
            pl.BlockSpec((_RB, F), lambda i: (i, 0)),
            pl.BlockSpec((F, H), lambda i: (0, 0)),
            pl.BlockSpec((_RB, 1), lambda i: (i, 0)),
            pl.BlockSpec((_RB, 1), lambda i: (i, 0)),
        ],
        out_specs=[
            pl.BlockSpec((_RB, H), lambda i: (i, 0)),
            pl.BlockSpec((_RB, 1), lambda i: (i, 0)),
        ],
        out_shape=[
            jax.ShapeDtypeStruct((N, H), jnp.float32),
            jax.ShapeDtypeStruct((N, 1), jnp.float32),
        ],
    )(x, w1, dega, degb)


def _tc_b_body(aa_ref, ab_ref, hs_ref, dinv_ref, b1_ref, w2_ref, out_ref):
    dinv = dinv_ref[...]
    pre = dinv * (aa_ref[...] + ab_ref[...] + hs_ref[...]) + b1_ref[...]
    r = jnp.maximum(pre, 0.0)
    h2 = jnp.dot(r, w2_ref[...], preferred_element_type=jnp.float32)
    out_ref[...] = h2 * dinv


def _tc_b(agg_a, agg_b, hs1, dinv, b1, w2):
    return pl.pallas_call(
        _tc_b_body,
        grid=(N // _RB,),
        in_specs=[
            pl.BlockSpec((_RB, H), lambda i: (i, 0)),
            pl.BlockSpec((_RB, H), lambda i: (i, 0)),
            pl.BlockSpec((_RB, H), lambda i: (i, 0)),
            pl.BlockSpec((_RB, 1), lambda i: (i, 0)),
            pl.BlockSpec((1, H), lambda i: (0, 0)),
            pl.BlockSpec((H, EMB), lambda i: (0, 0)),
        ],
        out_specs=pl.BlockSpec((_RB, EMB), lambda i: (i, 0)),
        out_shape=jax.ShapeDtypeStruct((N, EMB), jnp.float32),
    )(agg_a, agg_b, hs1, dinv, b1, w2)


def _tc_c_body(aa_ref, ab_ref, hs_ref, dinv_ref, b2_ref, out_ref):
    out_ref[...] = (dinv_ref[...] *
                    (aa_ref[...] + ab_ref[...] + hs_ref[...]) + b2_ref[...])


def _tc_c(agg_a, agg_b, hs2, dinv, b2):
    return pl.pallas_call(
        _tc_c_body,
        grid=(N // _RB,),
        in_specs=[
            pl.BlockSpec((_RB, EMB), lambda i: (i, 0)),
            pl.BlockSpec((_RB, EMB), lambda i: (i, 0)),
            pl.BlockSpec((_RB, EMB), lambda i: (i, 0)),
            pl.BlockSpec((_RB, 1), lambda i: (i, 0)),
            pl.BlockSpec((1, EMB), lambda i: (0, 0)),
        ],
        out_specs=pl.BlockSpec((_RB, EMB), lambda i: (i, 0)),
        out_shape=jax.ShapeDtypeStruct((N, EMB), jnp.float32),
    )(agg_a, agg_b, hs2, dinv, b2)


def kernel(x, ei, W1, b1, W2, b2):
    ei = ei.astype(jnp.int32)
    pad = E_PAD - E
    row_r = jnp.concatenate(
        [ei[0], jnp.zeros((pad,), jnp.int32)]).reshape(NW * G, B)
    col_r = jnp.concatenate(
        [ei[1], jnp.full((pad,), N, jnp.int32)]).reshape(NW * G, B)

    ones16 = jnp.ones((B, 16), jnp.float32)
    zeros16 = jnp.zeros((N_ACC, 16), jnp.float32)
    zeros32 = jnp.zeros((N_ACC, H), jnp.float32)

    deg2d = _deg_kernel(col_r, ones16, zeros16)
    dega = deg2d[0, :N, 0:1]
    degb = deg2d[1, :N, 0:1]

    hs1, dinv = _tc_a(x, W1, dega, degb)

    agg1 = _agg32(row_r, col_r, hs1, zeros32)
    hs2 = _tc_b(agg1[0, :N], agg1[1, :N], hs1, dinv, b1.reshape(1, H), W2)

    agg2 = _agg16(row_r, col_r, hs2, zeros16)
    out = _tc_c(agg2[0, :N], agg2[1, :N], hs2, dinv, b2.reshape(1, EMB))
    return out


# speedup vs baseline: 1.8723x; 1.1628x over previous
import functools

import jax
import jax.numpy as jnp
from jax import lax
from jax.experimental import pallas as pl
from jax.experimental.pallas import tpu as pltpu
from jax.experimental.pallas import tpu_sc as plsc

N = 10000
E = 320000
F = 128
H = 32
EMB = 16

NC = 2
NS = 16
NW = NC * NS
B = 128
G = 80
E_PAD = NW * G * B
NBUF = 4

N_ACC = 10112
ZCHUNK = N_ACC // NS

_mesh = plsc.VectorSubcoreMesh(core_axis_name="c", subcore_axis_name="s")
_sc_params = pltpu.CompilerParams(use_tc_tiling_on_sc=False)


def _fill_vmem(ref, rows, width, value):
    vec = jnp.full((16,), value, jnp.float32)

    @pl.loop(0, rows)
    def _(r):
        for j in range(width // 16):
            ref[r, pl.ds(j * 16, 16)] = vec


def _zero_shared(shared, zsrc, s, width):
    _fill_vmem(zsrc, B, width, 0.0)
    for i in range(ZCHUNK // B):
        pltpu.sync_copy(zsrc, shared.at[pl.ds(s * ZCHUNK + i * B, B)])
    rem = ZCHUNK % B
    if rem:
        pltpu.sync_copy(
            zsrc.at[pl.ds(0, rem)],
            shared.at[pl.ds(s * ZCHUNK + (ZCHUNK // B) * B, rem)])


def _deg_body(col_hbm, out_hbm, colv, onesv, zsrc, shared, ssems):
    c = lax.axis_index("c")
    s = lax.axis_index("s")
    wid = c * NS + s
    _zero_shared(shared, zsrc, s, 16)
    _fill_vmem(onesv, B, 16, 1.0)
    pltpu.sync_copy(col_hbm.at[pl.ds(wid * G, G)], colv)
    plsc.subcore_barrier()

    @pl.loop(0, G, step=NBUF)
    def _(g0):
        for b in range(NBUF):
            g = g0 + b

            @pl.when(g >= NBUF)
            def _drain():
                pltpu.make_async_copy(onesv, shared.at[colv.at[g - NBUF]],
                                      ssems.at[b]).wait()

            pltpu.async_copy(onesv, shared.at[colv.at[g]], ssems.at[b],
                             add=True)

    for b in range(NBUF):
        pltpu.make_async_copy(onesv, shared.at[colv.at[G - NBUF + b]],
                              ssems.at[b]).wait()

    plsc.subcore_barrier()
    pltpu.sync_copy(shared.at[pl.ds(s * ZCHUNK, ZCHUNK)],
                    out_hbm.at[c, pl.ds(s * ZCHUNK, ZCHUNK)])


_deg_kernel = functools.partial(
    pl.kernel,
    out_type=jax.ShapeDtypeStruct((NC, N_ACC, 16), jnp.float32),
    mesh=_mesh,
    compiler_params=_sc_params,
    scratch_types=[
        pltpu.VMEM((G, B), jnp.int32),
        pltpu.VMEM((B, 16), jnp.float32),
        pltpu.VMEM((B, 16), jnp.float32),
        pltpu.VMEM_SHARED((N_ACC, 16), jnp.float32),
        pltpu.SemaphoreType.DMA((NBUF,)),
    ],
)(_deg_body)


def _agg_body(row_hbm, col_hbm, hs_hbm, out_hbm,
              rowv, colv, bufs, shared, gsems, ssems, table=None):
    c = lax.axis_index("c")
    s = lax.axis_index("s")
    wid = c * NS + s
    D = bufs.shape[2]
    _zero_shared(shared, bufs.at[0], s, D)
    if table is not None:
        pltpu.sync_copy(hs_hbm.at[pl.ds(s * (N // NS), N // NS)],
                        table.at[pl.ds(s * (N // NS), N // NS)])
        src = table
    else:
        src = hs_hbm
    pltpu.sync_copy(row_hbm.at[pl.ds(wid * G, G)], rowv)
    pltpu.sync_copy(col_hbm.at[pl.ds(wid * G, G)], colv)
    plsc.subcore_barrier()

    NB2 = 2 * NBUF
    for b in range(NBUF):
        pltpu.async_copy(src.at[rowv.at[b]], bufs.at[b], gsems.at[b])

    @pl.loop(0, G, step=NB2)
    def _(g0):
        for db in range(NB2):
            g = g0 + db
            b = db
            pltpu.make_async_copy(src.at[rowv.at[g]], bufs.at[b],
                                  gsems.at[b]).wait()
            pltpu.async_copy(bufs.at[b], shared.at[colv.at[g]], ssems.at[b],
                             add=True)
            gn = g + NBUF
            bn = (db + NBUF) % NB2

            @pl.when(gn < G)
            def _issue():
                @pl.when(gn >= NB2)
                def _free():
                    pltpu.make_async_copy(bufs.at[bn],
                                          shared.at[colv.at[gn - NB2]],
                                          ssems.at[bn]).wait()

                pltpu.async_copy(src.at[rowv.at[gn]], bufs.at[bn],
                                 gsems.at[bn])

    for b in range(NB2):
        g_last = G - NB2 + b
        pltpu.make_async_copy(bufs.at[b], shared.at[colv.at[g_last]],
                              ssems.at[b]).wait()

    plsc.subcore_barrier()
    pltpu.sync_copy(shared.at[pl.ds(s * ZCHUNK, ZCHUNK)],
                    out_hbm.at[c, pl.ds(s * ZCHUNK, ZCHUNK)])


def _make_agg(D, spmem_table):
    scratch = [
        pltpu.VMEM((G, B), jnp.int32),
        pltpu.VMEM((G, B), jnp.int32),
        pltpu.VMEM((2 * NBUF, B, D), jnp.float32),
        pltpu.VMEM_SHARED((N_ACC, D), jnp.float32),
        pltpu.SemaphoreType.DMA((2 * NBUF,)),
        pltpu.SemaphoreType.DMA((2 * NBUF,)),
    ]
    if spmem_table:
        scratch.append(pltpu.VMEM_SHARED((N, D), jnp.float32))
    return functools.partial(
        pl.kernel,
        out_type=jax.ShapeDtypeStruct((NC, N_ACC, D), jnp.float32),
        mesh=_mesh,
        compiler_params=_sc_params,
        scratch_types=scratch,
    )(_agg_body)


_agg32 = _make_agg(H, True)
_agg16 = _make_agg(EMB, True)

def _tc_a_body(x_ref, w_ref, deg_ref, hs_ref, dinv_ref):
    deg = deg_ref[0, :N, 0:1] + deg_ref[1, :N, 0:1] + 1.0
    dinv = lax.rsqrt(deg)
    h = jnp.dot(x_ref[...], w_ref[...], preferred_element_type=jnp.float32)
    hs_ref[...] = h * dinv
    dinv_ref[...] = dinv


def _tc_a(x, w1, deg2d):
    return pl.pallas_call(
        _tc_a_body,
        out_shape=[
            jax.ShapeDtypeStruct((N, H), jnp.float32),
            jax.ShapeDtypeStruct((N, 1), jnp.float32),
        ],
    )(x, w1, deg2d)


def _tc_b_body(agg_ref, hs_ref, dinv_ref, b1_ref, w2_ref, out_ref):
    dinv = dinv_ref[...]
    pre = (dinv * (agg_ref[0, :N] + agg_ref[1, :N] + hs_ref[...])
           + b1_ref[...])
    r = jnp.maximum(pre, 0.0)
    h2 = jnp.dot(r, w2_ref[...], preferred_element_type=jnp.float32)
    out_ref[...] = h2 * dinv


def _tc_b(agg, hs1, dinv, b1, w2):
    return pl.pallas_call(
        _tc_b_body,
        out_shape=jax.ShapeDtypeStruct((N, EMB), jnp.float32),
    )(agg, hs1, dinv, b1, w2)


def _tc_c_body(agg_ref, hs_ref, dinv_ref, b2_ref, out_ref):
    out_ref[...] = (dinv_ref[...] *
                    (agg_ref[0, :N] + agg_ref[1, :N] + hs_ref[...])
                    + b2_ref[...])


def _tc_c(agg, hs2, dinv, b2):
    return pl.pallas_call(
        _tc_c_body,
        out_shape=jax.ShapeDtypeStruct((N, EMB), jnp.float32),
    )(agg, hs2, dinv, b2)


def kernel(x, ei, W1, b1, W2, b2):
    ei = ei.astype(jnp.int32)
    pad = E_PAD - E
    row_r = jnp.concatenate(
        [ei[0], jnp.zeros((pad,), jnp.int32)]).reshape(NW * G, B)
    col_r = jnp.concatenate(
        [ei[1], jnp.full((pad,), N, jnp.int32)]).reshape(NW * G, B)

    deg2d = _deg_kernel(col_r)
    hs1, dinv = _tc_a(x, W1, deg2d)

    agg1 = _agg32(row_r, col_r, hs1)
    hs2 = _tc_b(agg1, hs1, dinv, b1.reshape(1, H), W2)

    agg2 = _agg16(row_r, col_r, hs2)
    out = _tc_c(agg2, hs2, dinv, b2.reshape(1, EMB))
    return out


# speedup vs baseline: 2.0003x; 1.0683x over previous
import functools

import jax
import jax.numpy as jnp
from jax import lax
from jax.experimental import pallas as pl
from jax.experimental.pallas import tpu as pltpu
from jax.experimental.pallas import tpu_sc as plsc

N = 10000
E = 320000
F = 128
H = 32
EMB = 16

NC = 2
NS = 16
NW = NC * NS
B = 128
G = 80
E_PAD = NW * G * B
NBUF = 4

N_ACC = 10112
ZCHUNK = N_ACC // NS

_mesh = plsc.VectorSubcoreMesh(core_axis_name="c", subcore_axis_name="s")
_sc_params = pltpu.CompilerParams(use_tc_tiling_on_sc=False)


def _fill_vmem(ref, rows, width, value):
    vec = jnp.full((16,), value, jnp.float32)

    @pl.loop(0, rows)
    def _(r):
        for j in range(width // 16):
            ref[r, pl.ds(j * 16, 16)] = vec


def _zero_shared(shared, zsrc, s, width):
    _fill_vmem(zsrc, B, width, 0.0)
    for i in range(ZCHUNK // B):
        pltpu.sync_copy(zsrc, shared.at[pl.ds(s * ZCHUNK + i * B, B)])
    rem = ZCHUNK % B
    if rem:
        pltpu.sync_copy(
            zsrc.at[pl.ds(0, rem)],
            shared.at[pl.ds(s * ZCHUNK + (ZCHUNK // B) * B, rem)])


EPT = E // NW
FULLG = EPT // B


def _deg_body(ei_hbm, out_hbm, row_out, col_out,
              row1v, col1v, rowv, colv, onesv, zsrc, shared, ssems):
    c = lax.axis_index("c")
    s = lax.axis_index("s")
    wid = c * NS + s
    _zero_shared(shared, zsrc, s, 16)
    _fill_vmem(onesv, B, 16, 1.0)
    pltpu.sync_copy(ei_hbm.at[0, pl.ds(wid * EPT, EPT)], row1v)
    pltpu.sync_copy(ei_hbm.at[1, pl.ds(wid * EPT, EPT)], col1v)

    @pl.loop(0, FULLG)
    def _(g):
        for k in range(B // 16):
            src = pl.ds(g * B + k * 16, 16)
            rowv[g, pl.ds(k * 16, 16)] = row1v[src]
            colv[g, pl.ds(k * 16, 16)] = col1v[src]

    zvec = jnp.zeros((16,), jnp.int32)
    nvec = jnp.full((16,), N, jnp.int32)
    tail = EPT - FULLG * B
    for k in range(B // 16):
        off = pl.ds(k * 16, 16)
        if k * 16 < tail:
            src = pl.ds(FULLG * B + k * 16, 16)
            rowv[FULLG, off] = row1v[src]
            colv[FULLG, off] = col1v[src]
        else:
            rowv[FULLG, off] = zvec
            colv[FULLG, off] = nvec
    for g in range(FULLG + 1, G):
        for k in range(B // 16):
            off = pl.ds(k * 16, 16)
            rowv[g, off] = zvec
            colv[g, off] = nvec

    pltpu.sync_copy(rowv, row_out.at[pl.ds(wid * G, G)])
    pltpu.sync_copy(colv, col_out.at[pl.ds(wid * G, G)])
    plsc.subcore_barrier()

    @pl.loop(0, G, step=NBUF)
    def _(g0):
        for b in range(NBUF):
            g = g0 + b

            @pl.when(g >= NBUF)
            def _drain():
                pltpu.make_async_copy(onesv, shared.at[colv.at[g - NBUF]],
                                      ssems.at[b]).wait()

            pltpu.async_copy(onesv, shared.at[colv.at[g]], ssems.at[b],
                             add=True)

    for b in range(NBUF):
        pltpu.make_async_copy(onesv, shared.at[colv.at[G - NBUF + b]],
                              ssems.at[b]).wait()

    plsc.subcore_barrier()
    pltpu.sync_copy(shared.at[pl.ds(s * ZCHUNK, ZCHUNK)],
                    out_hbm.at[c, pl.ds(s * ZCHUNK, ZCHUNK)])


_deg_kernel = functools.partial(
    pl.kernel,
    out_type=[
        jax.ShapeDtypeStruct((NC, N_ACC, 16), jnp.float32),
        jax.ShapeDtypeStruct((NW * G, B), jnp.int32),
        jax.ShapeDtypeStruct((NW * G, B), jnp.int32),
    ],
    mesh=_mesh,
    compiler_params=_sc_params,
    scratch_types=[
        pltpu.VMEM((EPT,), jnp.int32),
        pltpu.VMEM((EPT,), jnp.int32),
        pltpu.VMEM((G, B), jnp.int32),
        pltpu.VMEM((G, B), jnp.int32),
        pltpu.VMEM((B, 16), jnp.float32),
        pltpu.VMEM((B, 16), jnp.float32),
        pltpu.VMEM_SHARED((N_ACC, 16), jnp.float32),
        pltpu.SemaphoreType.DMA((NBUF,)),
    ],
)(_deg_body)


def _agg_body(row_hbm, col_hbm, hs_hbm, out_hbm,
              rowv, colv, bufs, shared, gsems, ssems, table=None):
    c = lax.axis_index("c")
    s = lax.axis_index("s")
    wid = c * NS + s
    D = bufs.shape[2]
    _zero_shared(shared, bufs.at[0], s, D)
    if table is not None:
        pltpu.sync_copy(hs_hbm.at[pl.ds(s * (N // NS), N // NS)],
                        table.at[pl.ds(s * (N // NS), N // NS)])
        src = table
    else:
        src = hs_hbm
    pltpu.sync_copy(row_hbm.at[pl.ds(wid * G, G)], rowv)
    pltpu.sync_copy(col_hbm.at[pl.ds(wid * G, G)], colv)
    plsc.subcore_barrier()

    NB2 = 2 * NBUF
    for b in range(NBUF):
        pltpu.async_copy(src.at[rowv.at[b]], bufs.at[b], gsems.at[b])

    @pl.loop(0, G, step=NB2)
    def _(g0):
        for db in range(NB2):
            g = g0 + db
            b = db
            pltpu.make_async_copy(src.at[rowv.at[g]], bufs.at[b],
                                  gsems.at[b]).wait()
            pltpu.async_copy(bufs.at[b], shared.at[colv.at[g]], ssems.at[b],
                             add=True)
            gn = g + NBUF
            bn = (db + NBUF) % NB2

            @pl.when(gn < G)
            def _issue():
                @pl.when(gn >= NB2)
                def _free():
                    pltpu.make_async_copy(bufs.at[bn],
                                          shared.at[colv.at[gn - NB2]],
                                          ssems.at[bn]).wait()

                pltpu.async_copy(src.at[rowv.at[gn]], bufs.at[bn],
                                 gsems.at[bn])

    for b in range(NB2):
        g_last = G - NB2 + b
        pltpu.make_async_copy(bufs.at[b], shared.at[colv.at[g_last]],
                              ssems.at[b]).wait()

    plsc.subcore_barrier()
    pltpu.sync_copy(shared.at[pl.ds(s * ZCHUNK, ZCHUNK)],
                    out_hbm.at[c, pl.ds(s * ZCHUNK, ZCHUNK)])


def _make_agg(D, spmem_table):
    scratch = [
        pltpu.VMEM((G, B), jnp.int32),
        pltpu.VMEM((G, B), jnp.int32),
        pltpu.VMEM((2 * NBUF, B, D), jnp.float32),
        pltpu.VMEM_SHARED((N_ACC, D), jnp.float32),
        pltpu.SemaphoreType.DMA((2 * NBUF,)),
        pltpu.SemaphoreType.DMA((2 * NBUF,)),
    ]
    if spmem_table:
        scratch.append(pltpu.VMEM_SHARED((N, D), jnp.float32))
    return functools.partial(
        pl.kernel,
        out_type=jax.ShapeDtypeStruct((NC, N_ACC, D), jnp.float32),
        mesh=_mesh,
        compiler_params=_sc_params,
        scratch_types=scratch,
    )(_agg_body)


_agg32 = _make_agg(H, True)
_agg16 = _make_agg(EMB, True)

def _tc_a_body(x_ref, w_ref, deg_ref, hs_ref, dinv_ref):
    deg = deg_ref[0, :N, 0:1] + deg_ref[1, :N, 0:1] + 1.0
    dinv = lax.rsqrt(deg)
    h = jnp.dot(x_ref[...], w_ref[...], preferred_element_type=jnp.float32)
    hs_ref[...] = h * dinv
    dinv_ref[...] = dinv


def _tc_a(x, w1, deg2d):
    return pl.pallas_call(
        _tc_a_body,
        out_shape=[
            jax.ShapeDtypeStruct((N, H), jnp.float32),
            jax.ShapeDtypeStruct((N, 1), jnp.float32),
        ],
    )(x, w1, deg2d)


def _tc_b_body(agg_ref, hs_ref, dinv_ref, b1_ref, w2_ref, out_ref):
    dinv = dinv_ref[...]
    pre = (dinv * (agg_ref[0, :N] + agg_ref[1, :N] + hs_ref[...])
           + b1_ref[...])
    r = jnp.maximum(pre, 0.0)
    h2 = jnp.dot(r, w2_ref[...], preferred_element_type=jnp.float32)
    out_ref[...] = h2 * dinv


def _tc_b(agg, hs1, dinv, b1, w2):
    return pl.pallas_call(
        _tc_b_body,
        out_shape=jax.ShapeDtypeStruct((N, EMB), jnp.float32),
    )(agg, hs1, dinv, b1, w2)


def _tc_c_body(agg_ref, hs_ref, dinv_ref, b2_ref, out_ref):
    out_ref[...] = (dinv_ref[...] *
                    (agg_ref[0, :N] + agg_ref[1, :N] + hs_ref[...])
                    + b2_ref[...])


def _tc_c(agg, hs2, dinv, b2):
    return pl.pallas_call(
        _tc_c_body,
        out_shape=jax.ShapeDtypeStruct((N, EMB), jnp.float32),
    )(agg, hs2, dinv, b2)


def kernel(x, ei, W1, b1, W2, b2):
    ei = ei.astype(jnp.int32)
    deg2d, row_r, col_r = _deg_kernel(ei)
    hs1, dinv = _tc_a(x, W1, deg2d)

    agg1 = _agg32(row_r, col_r, hs1)
    hs2 = _tc_b(agg1, hs1, dinv, b1.reshape(1, H), W2)

    agg2 = _agg16(row_r, col_r, hs2)
    out = _tc_c(agg2, hs2, dinv, b2.reshape(1, EMB))
    return out


# speedup vs baseline: 2.1057x; 1.0527x over previous
import functools

import jax
import jax.numpy as jnp
from jax import lax
from jax.experimental import pallas as pl
from jax.experimental.pallas import tpu as pltpu
from jax.experimental.pallas import tpu_sc as plsc

N = 10000
E = 320000
F = 128
H = 32
EMB = 16

NC = 2
NS = 16
NW = NC * NS
B = 128
G = 80
E_PAD = NW * G * B
NBUF = 5

N_ACC = 10112
ZCHUNK = N_ACC // NS

_mesh = plsc.VectorSubcoreMesh(core_axis_name="c", subcore_axis_name="s")
_sc_params = pltpu.CompilerParams(use_tc_tiling_on_sc=False)


def _fill_vmem(ref, rows, width, value):
    vec = jnp.full((16,), value, jnp.float32)

    @pl.loop(0, rows)
    def _(r):
        for j in range(width // 16):
            ref[r, pl.ds(j * 16, 16)] = vec


def _zero_shared_start(shared, zsrc, s, psem):
    width = zsrc.shape[1]
    _fill_vmem(zsrc, B, width, 0.0)
    for i in range(ZCHUNK // B):
        pltpu.async_copy(zsrc, shared.at[pl.ds(s * ZCHUNK + i * B, B)],
                         psem.at[i])
    rem = ZCHUNK % B
    if rem:
        pltpu.async_copy(
            zsrc.at[pl.ds(0, rem)],
            shared.at[pl.ds(s * ZCHUNK + (ZCHUNK // B) * B, rem)],
            psem.at[ZCHUNK // B])


def _zero_shared_wait(shared, zsrc, s, psem):
    for i in range(ZCHUNK // B):
        pltpu.make_async_copy(zsrc, shared.at[pl.ds(s * ZCHUNK + i * B, B)],
                              psem.at[i]).wait()
    rem = ZCHUNK % B
    if rem:
        pltpu.make_async_copy(
            zsrc.at[pl.ds(0, rem)],
            shared.at[pl.ds(s * ZCHUNK + (ZCHUNK // B) * B, rem)],
            psem.at[ZCHUNK // B]).wait()


EPT = E // NW
FULLG = EPT // B


def _deg_body(ei_hbm, out_hbm, row_out, col_out,
              row1v, col1v, rowv, colv, onesv, zsrc, shared, ssems, psem):
    c = lax.axis_index("c")
    s = lax.axis_index("s")
    wid = c * NS + s
    era = pltpu.async_copy(ei_hbm.at[0, pl.ds(wid * EPT, EPT)], row1v,
                           psem.at[6])
    eca = pltpu.async_copy(ei_hbm.at[1, pl.ds(wid * EPT, EPT)], col1v,
                           psem.at[7])
    _zero_shared_start(shared, zsrc, s, psem)
    _fill_vmem(onesv, B, 16, 1.0)
    era.wait()
    eca.wait()

    @pl.loop(0, FULLG)
    def _(g):
        for k in range(B // 16):
            src = pl.ds(g * B + k * 16, 16)
            rowv[g, pl.ds(k * 16, 16)] = row1v[src]
            colv[g, pl.ds(k * 16, 16)] = col1v[src]

    zvec = jnp.zeros((16,), jnp.int32)
    nvec = jnp.full((16,), N, jnp.int32)
    tail = EPT - FULLG * B
    for k in range(B // 16):
        off = pl.ds(k * 16, 16)
        if k * 16 < tail:
            src = pl.ds(FULLG * B + k * 16, 16)
            rowv[FULLG, off] = row1v[src]
            colv[FULLG, off] = col1v[src]
        else:
            rowv[FULLG, off] = zvec
            colv[FULLG, off] = nvec
    for g in range(FULLG + 1, G):
        for k in range(B // 16):
            off = pl.ds(k * 16, 16)
            rowv[g, off] = zvec
            colv[g, off] = nvec

    ra = pltpu.async_copy(rowv, row_out.at[pl.ds(wid * G, G)], psem.at[6])
    ca = pltpu.async_copy(colv, col_out.at[pl.ds(wid * G, G)], psem.at[7])
    _zero_shared_wait(shared, zsrc, s, psem)
    ra.wait()
    ca.wait()
    plsc.subcore_barrier()

    @pl.loop(0, G, step=NBUF)
    def _(g0):
        for b in range(NBUF):
            g = g0 + b

            @pl.when(g >= NBUF)
            def _drain():
                pltpu.make_async_copy(onesv, shared.at[colv.at[g - NBUF]],
                                      ssems.at[b]).wait()

            pltpu.async_copy(onesv, shared.at[colv.at[g]], ssems.at[b],
                             add=True)

    for b in range(NBUF):
        pltpu.make_async_copy(onesv, shared.at[colv.at[G - NBUF + b]],
                              ssems.at[b]).wait()

    plsc.subcore_barrier()
    pltpu.sync_copy(shared.at[pl.ds(s * ZCHUNK, ZCHUNK)],
                    out_hbm.at[c, pl.ds(s * ZCHUNK, ZCHUNK)])


_deg_kernel = functools.partial(
    pl.kernel,
    out_type=[
        jax.ShapeDtypeStruct((NC, N_ACC, 16), jnp.float32),
        jax.ShapeDtypeStruct((NW * G, B), jnp.int32),
        jax.ShapeDtypeStruct((NW * G, B), jnp.int32),
    ],
    mesh=_mesh,
    compiler_params=_sc_params,
    scratch_types=[
        pltpu.VMEM((EPT,), jnp.int32),
        pltpu.VMEM((EPT,), jnp.int32),
        pltpu.VMEM((G, B), jnp.int32),
        pltpu.VMEM((G, B), jnp.int32),
        pltpu.VMEM((B, 16), jnp.float32),
        pltpu.VMEM((B, 16), jnp.float32),
        pltpu.VMEM_SHARED((N_ACC, 16), jnp.float32),
        pltpu.SemaphoreType.DMA((NBUF,)),
        pltpu.SemaphoreType.DMA((8,)),
    ],
)(_deg_body)


def _agg_body(row_hbm, col_hbm, hs_hbm, out_hbm,
              rowv, colv, bufs, shared, gsems, ssems, psem, table=None):
    c = lax.axis_index("c")
    s = lax.axis_index("s")
    wid = c * NS + s
    ra = pltpu.async_copy(row_hbm.at[pl.ds(wid * G, G)], rowv, psem.at[6])
    ca = pltpu.async_copy(col_hbm.at[pl.ds(wid * G, G)], colv, psem.at[7])
    ta = pltpu.async_copy(hs_hbm.at[pl.ds(s * (N // NS), N // NS)],
                          table.at[pl.ds(s * (N // NS), N // NS)],
                          psem.at[5])
    _zero_shared_start(shared, bufs.at[0], s, psem)
    src = table
    _zero_shared_wait(shared, bufs.at[0], s, psem)
    ta.wait()
    ra.wait()
    ca.wait()
    plsc.subcore_barrier()

    NB2 = 2 * NBUF
    for b in range(NBUF):
        pltpu.async_copy(src.at[rowv.at[b]], bufs.at[b], gsems.at[b])

    @pl.loop(0, G, step=NB2)
    def _(g0):
        for db in range(NB2):
            g = g0 + db
            b = db
            pltpu.make_async_copy(src.at[rowv.at[g]], bufs.at[b],
                                  gsems.at[b]).wait()
            pltpu.async_copy(bufs.at[b], shared.at[colv.at[g]], ssems.at[b],
                             add=True)
            gn = g + NBUF
            bn = (db + NBUF) % NB2

            @pl.when(gn < G)
            def _issue():
                @pl.when(gn >= NB2)
                def _free():
                    pltpu.make_async_copy(bufs.at[bn],
                                          shared.at[colv.at[gn - NB2]],
                                          ssems.at[bn]).wait()

                pltpu.async_copy(src.at[rowv.at[gn]], bufs.at[bn],
                                 gsems.at[bn])

    for b in range(NB2):
        g_last = G - NB2 + b
        pltpu.make_async_copy(bufs.at[b], shared.at[colv.at[g_last]],
                              ssems.at[b]).wait()

    plsc.subcore_barrier()
    pltpu.sync_copy(shared.at[pl.ds(s * ZCHUNK, ZCHUNK)],
                    out_hbm.at[c, pl.ds(s * ZCHUNK, ZCHUNK)])


def _make_agg(D):
    scratch = [
        pltpu.VMEM((G, B), jnp.int32),
        pltpu.VMEM((G, B), jnp.int32),
        pltpu.VMEM((2 * NBUF, B, D), jnp.float32),
        pltpu.VMEM_SHARED((N_ACC, D), jnp.float32),
        pltpu.SemaphoreType.DMA((2 * NBUF,)),
        pltpu.SemaphoreType.DMA((2 * NBUF,)),
        pltpu.SemaphoreType.DMA((8,)),
        pltpu.VMEM_SHARED((N, D), jnp.float32),
    ]
    return functools.partial(
        pl.kernel,
        out_type=jax.ShapeDtypeStruct((NC, N_ACC, D), jnp.float32),
        mesh=_mesh,
        compiler_params=_sc_params,
        scratch_types=scratch,
    )(_agg_body)


_agg32 = _make_agg(H)
_agg16 = _make_agg(EMB)

def _tc_a_body(x_ref, w_ref, deg_ref, hs_ref, dinv_ref):
    deg = deg_ref[0, :N, 0:1] + deg_ref[1, :N, 0:1] + 1.0
    dinv = lax.rsqrt(deg)
    h = jnp.dot(x_ref[...], w_ref[...], preferred_element_type=jnp.float32)
    hs_ref[...] = h * dinv
    dinv_ref[...] = dinv


def _tc_a(x, w1, deg2d):
    return pl.pallas_call(
        _tc_a_body,
        out_shape=[
            jax.ShapeDtypeStruct((N, H), jnp.float32),
            jax.ShapeDtypeStruct((N, 1), jnp.float32),
        ],
    )(x, w1, deg2d)


def _tc_b_body(agg_ref, hs_ref, dinv_ref, b1_ref, w2_ref, out_ref):
    dinv = dinv_ref[...]
    pre = (dinv * (agg_ref[0, :N] + agg_ref[1, :N] + hs_ref[...])
           + b1_ref[...])
    r = jnp.maximum(pre, 0.0)
    h2 = jnp.dot(r, w2_ref[...], preferred_element_type=jnp.float32)
    out_ref[...] = h2 * dinv


def _tc_b(agg, hs1, dinv, b1, w2):
    return pl.pallas_call(
        _tc_b_body,
        out_shape=jax.ShapeDtypeStruct((N, EMB), jnp.float32),
    )(agg, hs1, dinv, b1, w2)


def _tc_c_body(agg_ref, hs_ref, dinv_ref, b2_ref, out_ref):
    out_ref[...] = (dinv_ref[...] *
                    (agg_ref[0, :N] + agg_ref[1, :N] + hs_ref[...])
                    + b2_ref[...])


def _tc_c(agg, hs2, dinv, b2):
    return pl.pallas_call(
        _tc_c_body,
        out_shape=jax.ShapeDtypeStruct((N, EMB), jnp.float32),
    )(agg, hs2, dinv, b2)


def kernel(x, ei, W1, b1, W2, b2):
    ei = ei.astype(jnp.int32)
    deg2d, row_r, col_r = _deg_kernel(ei)
    hs1, dinv = _tc_a(x, W1, deg2d)

    agg1 = _agg32(row_r, col_r, hs1)
    hs2 = _tc_b(agg1, hs1, dinv, b1.reshape(1, H), W2)

    agg2 = _agg16(row_r, col_r, hs2)
    out = _tc_c(agg2, hs2, dinv, b2.reshape(1, EMB))
    return out
